# baseline (speedup 1.0000x reference)
import jax
import jax.numpy as jnp
from jax import lax
from jax.experimental import pallas as pl
from jax.experimental.pallas import tpu as pltpu
from jax.experimental.pallas import tpu_sc as plsc

_N = 10000
_E = 320000
_NC = 2
_NS = 16
_NW = _NC * _NS
_K = 125
_NB = 5
_LA = 2
_RPT = 624
_TAIL = _N - _RPT * _NS
_DDEG = 8

_EPW = _E // _NW
_NCH_D = _EPW // _K
_EPS = _E // _NS
_NCH_E = _EPS // _K

_F32 = jnp.float32
_SC_PARAMS = pltpu.CompilerParams(use_tc_tiling_on_sc=False)


def _mesh():
    return plsc.VectorSubcoreMesh(core_axis_name="c", subcore_axis_name="s")


def _zero_share(acc, zeros, sid):
    rbase = sid * _RPT
    pltpu.sync_copy(zeros.at[pl.ds(rbase, _RPT)], acc.at[pl.ds(rbase, _RPT)])

    @pl.when(sid == _NS - 1)
    def _():
        pltpu.sync_copy(zeros.at[pl.ds(_RPT * _NS, _TAIL)],
                        acc.at[pl.ds(_RPT * _NS, _TAIL)])


def _copy_out(acc, out, cid, sid):
    rbase = sid * _RPT
    pltpu.sync_copy(acc.at[pl.ds(rbase, _RPT)], out.at[cid, pl.ds(rbase, _RPT)])

    @pl.when(sid == _NS - 1)
    def _():
        pltpu.sync_copy(acc.at[pl.ds(_RPT * _NS, _TAIL)],
                        out.at[cid, pl.ds(_RPT * _NS, _TAIL)])


def _degree_body(e0, e1, e2, e3, ones4, zeros, out, acc, idxs,
                 o0, o1, o2, o3, *ssem):
    cid = lax.axis_index("c")
    sid = lax.axis_index("s")
    wid = cid * _NS + sid
    ones = (o0, o1, o2, o3)

    loads = [pltpu.async_copy(e.at[wid], idxs.at[j], ssem[0])
             for j, e in enumerate((e0, e1, e2, e3))]
    loads += [pltpu.async_copy(ones4.at[j], ones[j], ssem[1])
              for j in range(4)]
    _zero_share(acc, zeros, sid)
    for cp in loads:
        cp.wait()
    plsc.subcore_barrier()

    for j in range(4):
        def group(g, carry):
            for b in range(_NB):
                c = g * _NB + b
                if j == 0:
                    @pl.when(c >= _NB)
                    def _():
                        pltpu.make_async_copy(out.at[0, pl.ds(0, _K)],
                                              ones[0], ssem[b]).wait()
                else:
                    pltpu.make_async_copy(out.at[0, pl.ds(0, _K)],
                                          ones[0], ssem[b]).wait()
                pltpu.async_copy(ones[j], acc.at[idxs.at[j, c]], ssem[b],
                                 add=True)
            return carry

        lax.fori_loop(0, _NCH_D // _NB, group, 0)

    for b in range(_NB):
        pltpu.make_async_copy(out.at[0, pl.ds(0, _K)], ones[0], ssem[b]).wait()

    plsc.subcore_barrier()
    _copy_out(acc, out, cid, sid)


def _make_degrees():
    return pl.kernel(
        _degree_body,
        out_type=jax.ShapeDtypeStruct((_NC, _N, _DDEG), _F32),
        mesh=_mesh(),
        compiler_params=_SC_PARAMS,
        scratch_types=(
            [pltpu.VMEM_SHARED((_N, _DDEG), _F32),
             pltpu.VMEM((4, _NCH_D, _K), jnp.int32)]
            + [pltpu.VMEM((_K, _DDEG), _F32) for _ in range(4)]
            + [pltpu.SemaphoreType.DMA for _ in range(_NB)]
        ),
    )


def _make_edge_pass(dh, mode):
    nch = _NCH_E if mode == "split" else _NCH_D

    def body(table, srcx, dstx, zeros, out, acc, sidx, didx, *rest):
        rows = rest[0:_NB]
        gsem = rest[_NB:2 * _NB]
        ssem = rest[2 * _NB:3 * _NB]
        cid = lax.axis_index("c")
        sid = lax.axis_index("s")

        if mode == "split":
            ld_s = pltpu.async_copy(srcx.at[cid, sid], sidx, gsem[1])
            ld_d = pltpu.async_copy(dstx.at[sid], didx, gsem[2])
        else:
            wid = cid * _NS + sid
            ld_s = pltpu.async_copy(srcx.at[wid], sidx, gsem[1])
            ld_d = pltpu.async_copy(dstx.at[wid], didx, gsem[2])
        _zero_share(acc, zeros, sid)
        ld_s.wait()
        ld_d.wait()
        plsc.subcore_barrier()

        for c in range(_LA):
            pltpu.async_copy(table.at[sidx.at[c]], rows[c % _NB],
                             gsem[c % _NB])

        def group(g, carry):
            for b in range(_NB):
                c = g * _NB + b
                bg = (b + _LA) % _NB
                pltpu.make_async_copy(table.at[pl.ds(0, _K)], rows[b],
                                      gsem[b]).wait()
                pltpu.async_copy(rows[b], acc.at[didx.at[c]], ssem[b],
                                 add=True)

                @pl.when(jnp.logical_and(c + _LA < nch, c + _LA >= _NB))
                def _():
                    pltpu.make_async_copy(table.at[pl.ds(0, _K)], rows[bg],
                                          ssem[bg]).wait()

                @pl.when(c + _LA < nch)
                def _():
                    pltpu.async_copy(table.at[sidx.at[c + _LA]], rows[bg],
                                     gsem[bg])
            return carry

        lax.fori_loop(0, nch // _NB, group, 0)

        for b in range(_NB):
            pltpu.make_async_copy(table.at[pl.ds(0, _K)], rows[b],
                                  ssem[b]).wait()

        plsc.subcore_barrier()
        rbase = sid * _RPT
        if mode == "split":
            dsts = (out.at[pl.ds(rbase, _RPT), pl.ds(cid * dh, dh)],
                    out.at[pl.ds(_RPT * _NS, _TAIL), pl.ds(cid * dh, dh)])
        else:
            dsts = (out.at[cid, pl.ds(rbase, _RPT), pl.ds(0, dh)],
                    out.at[cid, pl.ds(_RPT * _NS, _TAIL), pl.ds(0, dh)])
        pltpu.sync_copy(acc.at[pl.ds(rbase, _RPT)], dsts[0])

        @pl.when(sid == _NS - 1)
        def _():
            pltpu.sync_copy(acc.at[pl.ds(_RPT * _NS, _TAIL)], dsts[1])

    out_shape = ((_N, 2 * dh) if mode == "split" else (_NC, _N, 128))
    return pl.kernel(
        body,
        out_type=jax.ShapeDtypeStruct(out_shape, _F32),
        mesh=_mesh(),
        compiler_params=_SC_PARAMS,
        scratch_types=(
            [pltpu.VMEM_SHARED((_N, dh), _F32),
             pltpu.VMEM((nch, _K), jnp.int32),
             pltpu.VMEM((nch, _K), jnp.int32)]
            + [pltpu.VMEM((_K, dh), _F32) for _ in range(_NB)]
            + [pltpu.SemaphoreType.DMA for _ in range(2 * _NB)]
        ),
    )


def _rs(cnt, j):
    return lax.rsqrt(jnp.maximum(jnp.transpose(cnt[j:j + 1, :]), 1.0))


def _cnts(cnt_ref):
    return cnt_ref[0] + cnt_ref[1]


def _tc_pre_body(x_ref, cnt_ref, w_ref, y_ref):
    cnt = _cnts(cnt_ref)
    y_ref[...] = jnp.dot(x_ref[...] * _rs(cnt, 0), w_ref[...],
                         preferred_element_type=_F32)


def _tc_mid_body(s1_ref, cnt_ref, b1_ref, w2_ref, y2_ref):
    cnt = _cnts(cnt_ref)
    h = jnp.maximum(s1_ref[...] * _rs(cnt, 1) + b1_ref[...], 0.0)
    y2 = jnp.dot(h * _rs(cnt, 2), w2_ref[...], preferred_element_type=_F32)
    y2_ref[:, pl.ds(0, y2.shape[1])] = y2


def _tc_post_body(s2_ref, cnt_ref, b2_ref, out_ref):
    cnt = _cnts(cnt_ref)
    d = out_ref.shape[1]
    s2 = s2_ref[0][:, :d] + s2_ref[1][:, :d]
    out_ref[...] = s2 * _rs(cnt, 3) + b2_ref[...]


def _split_edges(edge_index):
    src = edge_index[0].reshape(_NS, _NCH_E, _K)
    dst = edge_index[1].reshape(_NS, _NCH_E, _K)
    src4 = jnp.stack([2 * src, 2 * src + 1])
    return src4, dst


def _gcn(inputs, edge_index0, edge_index1, W1, b1, W2, b2):
    src0_d = edge_index0[0].reshape(_NW, _NCH_D, _K)
    dst0_d = edge_index0[1].reshape(_NW, _NCH_D, _K)
    src1_d = edge_index1[0].reshape(_NW, _NCH_D, _K)
    dst1_d = edge_index1[1].reshape(_NW, _NCH_D, _K)
    d_hid = W1.shape[1]
    d_out = W2.shape[1]
    src0, dst0 = _split_edges(edge_index0)
    src1f = src1_d * 2
    dst1f = dst1_d

    ones4 = jnp.broadcast_to(
        (jnp.arange(_DDEG)[None, None, :] ==
         jnp.arange(4)[:, None, None]).astype(_F32), (4, _K, _DDEG))
    zeros_deg = jnp.zeros((_N, _DDEG), _F32)
    zeros_edge = jnp.zeros((_N, d_hid // 2), _F32)

    cnt = _make_degrees()(src0_d, dst0_d, src1_d, dst1_d, ones4, zeros_deg)
    cntv = jnp.transpose(cnt, (0, 2, 1))[:, :4, :]

    y1 = pl.pallas_call(
        _tc_pre_body,
        out_shape=jax.ShapeDtypeStruct((_N, d_hid), _F32),
    )(inputs, cntv, W1)

    s1 = _make_edge_pass(d_hid // 2, "split")(
        y1.reshape(2 * _N, d_hid // 2), src0, dst0, zeros_edge)

    y2 = pl.pallas_call(
        _tc_mid_body,
        out_shape=jax.ShapeDtypeStruct((_N, d_hid), _F32),
    )(s1, cntv, b1.reshape(1, -1), W2)

    s2 = _make_edge_pass(d_out, "full")(
        y2.reshape(2 * _N, d_hid // 2), src1f, dst1f, zeros_edge)

    out = pl.pallas_call(
        _tc_post_body,
        out_shape=jax.ShapeDtypeStruct((_N, d_out), _F32),
    )(s2, cntv, b2.reshape(1, -1))

    return out


def kernel(inputs, edge_index0, edge_index1, W1, b1, W2, b2):
    return _gcn(inputs, edge_index0, edge_index1, W1, b1, W2, b2)


# speedup vs baseline: 1.0077x; 1.0077x over previous
import jax
import jax.numpy as jnp
from jax import lax
from jax.experimental import pallas as pl
from jax.experimental.pallas import tpu as pltpu
from jax.experimental.pallas import tpu_sc as plsc

_N = 10000
_E = 320000
_NC = 2
_NS = 16
_NW = _NC * _NS
_K = 125
_NB = 5
_LA = 2
_RPT = 624
_TAIL = _N - _RPT * _NS
_DDEG = 8

_EPW = _E // _NW
_NCH_D = _EPW // _K
_EPS = _E // _NS
_NCH_E = _EPS // _K

_F32 = jnp.float32
_SC_PARAMS = pltpu.CompilerParams(use_tc_tiling_on_sc=False)


def _mesh():
    return plsc.VectorSubcoreMesh(core_axis_name="c", subcore_axis_name="s")


def _zero_share(acc, zeros, sid):
    rbase = sid * _RPT
    pltpu.sync_copy(zeros.at[pl.ds(rbase, _RPT)], acc.at[pl.ds(rbase, _RPT)])

    @pl.when(sid == _NS - 1)
    def _():
        pltpu.sync_copy(zeros.at[pl.ds(_RPT * _NS, _TAIL)],
                        acc.at[pl.ds(_RPT * _NS, _TAIL)])


def _copy_out(acc, out, cid, sid):
    rbase = sid * _RPT
    pltpu.sync_copy(acc.at[pl.ds(rbase, _RPT)], out.at[cid, pl.ds(rbase, _RPT)])

    @pl.when(sid == _NS - 1)
    def _():
        pltpu.sync_copy(acc.at[pl.ds(_RPT * _NS, _TAIL)],
                        out.at[cid, pl.ds(_RPT * _NS, _TAIL)])


def _degree_body(e0, e1, e2, e3, ones4, zeros, out, acc, idxs,
                 o0, o1, o2, o3, *ssem):
    cid = lax.axis_index("c")
    sid = lax.axis_index("s")
    wid = cid * _NS + sid
    ones = (o0, o1, o2, o3)

    loads = [pltpu.async_copy(e.at[wid], idxs.at[j], ssem[0])
             for j, e in enumerate((e0, e1, e2, e3))]
    loads += [pltpu.async_copy(ones4.at[j], ones[j], ssem[1])
              for j in range(4)]
    _zero_share(acc, zeros, sid)
    for cp in loads:
        cp.wait()
    plsc.subcore_barrier()

    for j in range(4):
        def group(g, carry):
            for b in range(_NB):
                c = g * _NB + b
                if j == 0:
                    @pl.when(c >= _NB)
                    def _():
                        pltpu.make_async_copy(out.at[0, pl.ds(0, _K)],
                                              ones[0], ssem[b]).wait()
                else:
                    pltpu.make_async_copy(out.at[0, pl.ds(0, _K)],
                                          ones[0], ssem[b]).wait()
                pltpu.async_copy(ones[j], acc.at[idxs.at[j, c]], ssem[b],
                                 add=True)
            return carry

        lax.fori_loop(0, _NCH_D // _NB, group, 0)

    for b in range(_NB):
        pltpu.make_async_copy(out.at[0, pl.ds(0, _K)], ones[0], ssem[b]).wait()

    plsc.subcore_barrier()
    _copy_out(acc, out, cid, sid)


def _make_degrees():
    return pl.kernel(
        _degree_body,
        out_type=jax.ShapeDtypeStruct((_NC, _N, _DDEG), _F32),
        mesh=_mesh(),
        compiler_params=_SC_PARAMS,
        scratch_types=(
            [pltpu.VMEM_SHARED((_N, _DDEG), _F32),
             pltpu.VMEM((4, _NCH_D, _K), jnp.int32)]
            + [pltpu.VMEM((_K, _DDEG), _F32) for _ in range(4)]
            + [pltpu.SemaphoreType.DMA for _ in range(_NB)]
        ),
    )


def _make_edge_pass(dh, mode):
    nch = _NCH_E if mode == "split" else _NCH_D

    def body(table, srcx, dstx, zeros, out, acc, sidx, didx, *rest):
        rows = rest[0:_NB]
        gsem = rest[_NB:2 * _NB]
        ssem = rest[2 * _NB:3 * _NB]
        cid = lax.axis_index("c")
        sid = lax.axis_index("s")

        if mode == "split":
            ld_s = pltpu.async_copy(srcx.at[cid, sid], sidx, gsem[1])
            ld_d = pltpu.async_copy(dstx.at[sid], didx, gsem[2])
        else:
            wid = cid * _NS + sid
            ld_s = pltpu.async_copy(srcx.at[wid], sidx, gsem[1])
            ld_d = pltpu.async_copy(dstx.at[wid], didx, gsem[2])
        _zero_share(acc, zeros, sid)
        ld_s.wait()
        ld_d.wait()
        plsc.subcore_barrier()

        for c in range(_LA):
            pltpu.async_copy(table.at[sidx.at[c]], rows[c % _NB],
                             gsem[c % _NB])

        def group(g, carry):
            for b in range(_NB):
                c = g * _NB + b
                bg = (b + _LA) % _NB
                pltpu.make_async_copy(table.at[pl.ds(0, _K)], rows[b],
                                      gsem[b]).wait()
                pltpu.async_copy(rows[b], acc.at[didx.at[c]], ssem[b],
                                 add=True)

                @pl.when(jnp.logical_and(c + _LA < nch, c + _LA >= _NB))
                def _():
                    pltpu.make_async_copy(table.at[pl.ds(0, _K)], rows[bg],
                                          ssem[bg]).wait()

                @pl.when(c + _LA < nch)
                def _():
                    pltpu.async_copy(table.at[sidx.at[c + _LA]], rows[bg],
                                     gsem[bg])
            return carry

        lax.fori_loop(0, nch // _NB, group, 0)

        for b in range(_NB):
            pltpu.make_async_copy(table.at[pl.ds(0, _K)], rows[b],
                                  ssem[b]).wait()

        plsc.subcore_barrier()
        rbase = sid * _RPT
        if mode == "split":
            dsts = (out.at[pl.ds(rbase, _RPT), pl.ds(cid * dh, dh)],
                    out.at[pl.ds(_RPT * _NS, _TAIL), pl.ds(cid * dh, dh)])
        else:
            dsts = (out.at[cid, pl.ds(rbase, _RPT), pl.ds(0, dh)],
                    out.at[cid, pl.ds(_RPT * _NS, _TAIL), pl.ds(0, dh)])
        pltpu.sync_copy(acc.at[pl.ds(rbase, _RPT)], dsts[0])

        @pl.when(sid == _NS - 1)
        def _():
            pltpu.sync_copy(acc.at[pl.ds(_RPT * _NS, _TAIL)], dsts[1])

    out_shape = ((_N, 2 * dh) if mode == "split" else (_NC, _N, 128))
    return pl.kernel(
        body,
        out_type=jax.ShapeDtypeStruct(out_shape, _F32),
        mesh=_mesh(),
        compiler_params=_SC_PARAMS,
        scratch_types=(
            [pltpu.VMEM_SHARED((_N, dh), _F32),
             pltpu.VMEM((nch, _K), jnp.int32),
             pltpu.VMEM((nch, _K), jnp.int32)]
            + [pltpu.VMEM((_K, dh), _F32) for _ in range(_NB)]
            + [pltpu.SemaphoreType.DMA for _ in range(2 * _NB)]
        ),
    )


def _rs(cnt, j):
    return lax.rsqrt(jnp.maximum(cnt[:, j:j + 1], 1.0))


def _cnts(cnt_ref):
    return cnt_ref[0] + cnt_ref[1]


def _tc_pre_body(x_ref, cnt_ref, w_ref, y_ref, scl_ref):
    cnt = _cnts(cnt_ref)
    y_ref[...] = jnp.dot(x_ref[...] * _rs(cnt, 0), w_ref[...],
                         preferred_element_type=_F32)
    scl_ref[:, pl.ds(0, 4)] = jnp.concatenate(
        [_rs(cnt, j) for j in range(4)], axis=1)


def _tc_mid_body(s1_ref, scl_ref, b1_ref, w2_ref, y2_ref):
    h = jnp.maximum(s1_ref[...] * scl_ref[:, 1:2] + b1_ref[...], 0.0)
    y2 = jnp.dot(h * scl_ref[:, 2:3], w2_ref[...], preferred_element_type=_F32)
    y2_ref[:, pl.ds(0, y2.shape[1])] = y2


def _tc_post_body(s2_ref, scl_ref, b2_ref, out_ref):
    d = out_ref.shape[1]
    s2 = s2_ref[0][:, :d] + s2_ref[1][:, :d]
    out_ref[...] = s2 * scl_ref[:, 3:4] + b2_ref[...]


def _split_edges(edge_index):
    src = edge_index[0].reshape(_NS, _NCH_E, _K)
    dst = edge_index[1].reshape(_NS, _NCH_E, _K)
    src4 = jnp.stack([2 * src, 2 * src + 1])
    return src4, dst


def _gcn(inputs, edge_index0, edge_index1, W1, b1, W2, b2):
    src0_d = edge_index0[0].reshape(_NW, _NCH_D, _K)
    dst0_d = edge_index0[1].reshape(_NW, _NCH_D, _K)
    src1_d = edge_index1[0].reshape(_NW, _NCH_D, _K)
    dst1_d = edge_index1[1].reshape(_NW, _NCH_D, _K)
    d_hid = W1.shape[1]
    d_out = W2.shape[1]
    src0, dst0 = _split_edges(edge_index0)
    src1f = src1_d * 2
    dst1f = dst1_d

    ones4 = jnp.broadcast_to(
        (jnp.arange(_DDEG)[None, None, :] ==
         jnp.arange(4)[:, None, None]).astype(_F32), (4, _K, _DDEG))
    zeros_deg = jnp.zeros((_N, _DDEG), _F32)
    zeros_edge = jnp.zeros((_N, d_hid // 2), _F32)

    cnt = _make_degrees()(src0_d, dst0_d, src1_d, dst1_d, ones4, zeros_deg)
    cntv = cnt

    y1, scl = pl.pallas_call(
        _tc_pre_body,
        out_shape=(jax.ShapeDtypeStruct((_N, d_hid), _F32),
                   jax.ShapeDtypeStruct((_N, 128), _F32)),
    )(inputs, cntv, W1)

    s1 = _make_edge_pass(d_hid // 2, "split")(
        y1.reshape(2 * _N, d_hid // 2), src0, dst0, zeros_edge)

    y2 = pl.pallas_call(
        _tc_mid_body,
        out_shape=jax.ShapeDtypeStruct((_N, d_hid), _F32),
    )(s1, scl, b1.reshape(1, -1), W2)

    s2 = _make_edge_pass(d_out, "full")(
        y2.reshape(2 * _N, d_hid // 2), src1f, dst1f, zeros_edge)

    out = pl.pallas_call(
        _tc_post_body,
        out_shape=jax.ShapeDtypeStruct((_N, d_out), _F32),
    )(s2, scl, b2.reshape(1, -1))

    return out


def kernel(inputs, edge_index0, edge_index1, W1, b1, W2, b2):
    return _gcn(inputs, edge_index0, edge_index1, W1, b1, W2, b2)


# speedup vs baseline: 1.0134x; 1.0056x over previous
import jax
import jax.numpy as jnp
from jax import lax
from jax.experimental import pallas as pl
from jax.experimental.pallas import tpu as pltpu
from jax.experimental.pallas import tpu_sc as plsc

_N = 10000
_E = 320000
_NC = 2
_NS = 16
_NW = _NC * _NS
_K = 125
_NB = 5
_LA = 2
_RPT = 624
_TAIL = _N - _RPT * _NS
_DDEG = 8

_EPW = _E // _NW
_NCH_D = _EPW // _K
_EPS = _E // _NS
_NCH_E = _EPS // _K

_F32 = jnp.float32
_SC_PARAMS = pltpu.CompilerParams(use_tc_tiling_on_sc=False)


def _mesh():
    return plsc.VectorSubcoreMesh(core_axis_name="c", subcore_axis_name="s")


def _zero_share(acc, zeros, sid):
    rbase = sid * _RPT
    pltpu.sync_copy(zeros.at[pl.ds(rbase, _RPT)], acc.at[pl.ds(rbase, _RPT)])

    @pl.when(sid == _NS - 1)
    def _():
        pltpu.sync_copy(zeros.at[pl.ds(_RPT * _NS, _TAIL)],
                        acc.at[pl.ds(_RPT * _NS, _TAIL)])


def _copy_out(acc, out, cid, sid):
    rbase = sid * _RPT
    pltpu.sync_copy(acc.at[pl.ds(rbase, _RPT)], out.at[cid, pl.ds(rbase, _RPT)])

    @pl.when(sid == _NS - 1)
    def _():
        pltpu.sync_copy(acc.at[pl.ds(_RPT * _NS, _TAIL)],
                        out.at[cid, pl.ds(_RPT * _NS, _TAIL)])


def _degree_body(e0, e1, e2, e3, ones4, zeros, out, acc, idxs,
                 o0, o1, o2, o3, *ssem):
    cid = lax.axis_index("c")
    sid = lax.axis_index("s")
    wid = cid * _NS + sid
    ones = (o0, o1, o2, o3)

    loads = [pltpu.async_copy(e.at[wid], idxs.at[j], ssem[0])
             for j, e in enumerate((e0, e1, e2, e3))]
    loads += [pltpu.async_copy(ones4.at[j], ones[j], ssem[1])
              for j in range(4)]
    _zero_share(acc, zeros, sid)
    for cp in loads:
        cp.wait()
    plsc.subcore_barrier()

    for j in range(4):
        def group(g, carry):
            for b in range(_NB):
                c = g * _NB + b
                if j == 0:
                    @pl.when(c >= _NB)
                    def _():
                        pltpu.make_async_copy(out.at[0, pl.ds(0, _K)],
                                              ones[0], ssem[b]).wait()
                else:
                    pltpu.make_async_copy(out.at[0, pl.ds(0, _K)],
                                          ones[0], ssem[b]).wait()
                pltpu.async_copy(ones[j], acc.at[idxs.at[j, c]], ssem[b],
                                 add=True)
            return carry

        lax.fori_loop(0, _NCH_D // _NB, group, 0)

    for b in range(_NB):
        pltpu.make_async_copy(out.at[0, pl.ds(0, _K)], ones[0], ssem[b]).wait()

    plsc.subcore_barrier()
    _copy_out(acc, out, cid, sid)


def _make_degrees():
    return pl.kernel(
        _degree_body,
        out_type=jax.ShapeDtypeStruct((_NC, _N, _DDEG), _F32),
        mesh=_mesh(),
        compiler_params=_SC_PARAMS,
        scratch_types=(
            [pltpu.VMEM_SHARED((_N, _DDEG), _F32),
             pltpu.VMEM((4, _NCH_D, _K), jnp.int32)]
            + [pltpu.VMEM((_K, _DDEG), _F32) for _ in range(4)]
            + [pltpu.SemaphoreType.DMA for _ in range(_NB)]
        ),
    )


def _make_edge_pass(dh, mode):
    nch = _NCH_E if mode == "split" else _NCH_D

    def body(table, srcx, dstx, zeros, out, acc, sidx, didx, *rest):
        rows = rest[0:_NB]
        gsem = rest[_NB:2 * _NB]
        ssem = rest[2 * _NB:3 * _NB]
        cid = lax.axis_index("c")
        sid = lax.axis_index("s")

        if mode == "split":
            ld_s = pltpu.async_copy(srcx.at[cid, sid], sidx, gsem[1])
            ld_d = pltpu.async_copy(dstx.at[sid], didx, gsem[2])
        else:
            wid = cid * _NS + sid
            ld_s = pltpu.async_copy(srcx.at[wid], sidx, gsem[1])
            ld_d = pltpu.async_copy(dstx.at[wid], didx, gsem[2])
        _zero_share(acc, zeros, sid)
        ld_s.wait()
        ld_d.wait()
        plsc.subcore_barrier()

        for c in range(_LA):
            pltpu.async_copy(table.at[sidx.at[c]], rows[c % _NB],
                             gsem[c % _NB])

        def group(g, carry):
            for b in range(_NB):
                c = g * _NB + b
                bg = (b + _LA) % _NB
                pltpu.make_async_copy(table.at[pl.ds(0, _K)], rows[b],
                                      gsem[b]).wait()
                pltpu.async_copy(rows[b], acc.at[didx.at[c]], ssem[b],
                                 add=True)

                @pl.when(jnp.logical_and(c + _LA < nch, c + _LA >= _NB))
                def _():
                    pltpu.make_async_copy(table.at[pl.ds(0, _K)], rows[bg],
                                          ssem[bg]).wait()

                @pl.when(c + _LA < nch)
                def _():
                    pltpu.async_copy(table.at[sidx.at[c + _LA]], rows[bg],
                                     gsem[bg])
            return carry

        lax.fori_loop(0, nch // _NB, group, 0)

        for b in range(_NB):
            pltpu.make_async_copy(table.at[pl.ds(0, _K)], rows[b],
                                  ssem[b]).wait()

        plsc.subcore_barrier()
        rbase = sid * _RPT
        if mode == "split":
            dsts = (out.at[pl.ds(rbase, _RPT), pl.ds(cid * dh, dh)],
                    out.at[pl.ds(_RPT * _NS, _TAIL), pl.ds(cid * dh, dh)])
        else:
            dsts = (out.at[cid, pl.ds(rbase, _RPT), pl.ds(0, dh)],
                    out.at[cid, pl.ds(_RPT * _NS, _TAIL), pl.ds(0, dh)])
        pltpu.sync_copy(acc.at[pl.ds(rbase, _RPT)], dsts[0])

        @pl.when(sid == _NS - 1)
        def _():
            pltpu.sync_copy(acc.at[pl.ds(_RPT * _NS, _TAIL)], dsts[1])

    out_shape = ((_N, 2 * dh) if mode == "split" else (_NC, _N, 128))
    return pl.kernel(
        body,
        out_type=jax.ShapeDtypeStruct(out_shape, _F32),
        mesh=_mesh(),
        compiler_params=_SC_PARAMS,
        scratch_types=(
            [pltpu.VMEM_SHARED((_N, dh), _F32),
             pltpu.VMEM((nch, _K), jnp.int32),
             pltpu.VMEM((nch, _K), jnp.int32)]
            + [pltpu.VMEM((_K, dh), _F32) for _ in range(_NB)]
            + [pltpu.SemaphoreType.DMA for _ in range(2 * _NB)]
        ),
    )


def _rs(cnt, j):
    return lax.rsqrt(jnp.maximum(cnt[:, j:j + 1], 1.0))


def _cnts(cnt_ref):
    return cnt_ref[0] + cnt_ref[1]


def _tc_pre_body(x_ref, cnt_ref, w_ref, y_ref):
    cnt = _cnts(cnt_ref)
    y_ref[...] = jnp.dot(x_ref[...] * _rs(cnt, 0), w_ref[...],
                         preferred_element_type=_F32)


def _tc_mid_body(s1_ref, cnt_ref, b1_ref, w2_ref, y2_ref):
    cnt = _cnts(cnt_ref)
    h = jnp.maximum(s1_ref[...] * _rs(cnt, 1) + b1_ref[...], 0.0)
    y2 = jnp.dot(h * _rs(cnt, 2), w2_ref[...], preferred_element_type=_F32)
    y2_ref[:, pl.ds(0, y2.shape[1])] = y2


def _tc_post_body(s2_ref, cnt_ref, b2_ref, out_ref):
    cnt = _cnts(cnt_ref)
    d = out_ref.shape[1]
    s2 = s2_ref[0][:, :d] + s2_ref[1][:, :d]
    out_ref[...] = s2 * _rs(cnt, 3) + b2_ref[...]


def _split_edges(edge_index):
    src = edge_index[0].reshape(_NS, _NCH_E, _K)
    dst = edge_index[1].reshape(_NS, _NCH_E, _K)
    src4 = jnp.stack([2 * src, 2 * src + 1])
    return src4, dst


def _gcn(inputs, edge_index0, edge_index1, W1, b1, W2, b2):
    src0_d = edge_index0[0].reshape(_NW, _NCH_D, _K)
    dst0_d = edge_index0[1].reshape(_NW, _NCH_D, _K)
    src1_d = edge_index1[0].reshape(_NW, _NCH_D, _K)
    dst1_d = edge_index1[1].reshape(_NW, _NCH_D, _K)
    d_hid = W1.shape[1]
    d_out = W2.shape[1]
    src0, dst0 = _split_edges(edge_index0)
    src1f = src1_d * 2
    dst1f = dst1_d

    ones4 = jnp.broadcast_to(
        (jnp.arange(_DDEG)[None, None, :] ==
         jnp.arange(4)[:, None, None]).astype(_F32), (4, _K, _DDEG))
    zeros_deg = jnp.zeros((_N, _DDEG), _F32)
    zeros_edge = jnp.zeros((_N, d_hid // 2), _F32)

    cnt = _make_degrees()(src0_d, dst0_d, src1_d, dst1_d, ones4, zeros_deg)
    cntv = cnt

    y1 = pl.pallas_call(
        _tc_pre_body,
        out_shape=jax.ShapeDtypeStruct((_N, d_hid), _F32),
    )(inputs, cntv, W1)

    s1 = _make_edge_pass(d_hid // 2, "split")(
        y1.reshape(2 * _N, d_hid // 2), src0, dst0, zeros_edge)

    y2 = pl.pallas_call(
        _tc_mid_body,
        out_shape=jax.ShapeDtypeStruct((_N, d_hid), _F32),
    )(s1, cntv, b1.reshape(1, -1), W2)

    s2 = _make_edge_pass(d_out, "full")(
        y2.reshape(2 * _N, d_hid // 2), src1f, dst1f, zeros_edge)

    out = pl.pallas_call(
        _tc_post_body,
        out_shape=jax.ShapeDtypeStruct((_N, d_out), _F32),
    )(s2, cntv, b2.reshape(1, -1))

    return out


def kernel(inputs, edge_index0, edge_index1, W1, b1, W2, b2):
    return _gcn(inputs, edge_index0, edge_index1, W1, b1, W2, b2)


# speedup vs baseline: 1.0161x; 1.0026x over previous
import jax
import jax.numpy as jnp
from jax import lax
from jax.experimental import pallas as pl
from jax.experimental.pallas import tpu as pltpu
from jax.experimental.pallas import tpu_sc as plsc

_N = 10000
_E = 320000
_NC = 2
_NS = 16
_NW = _NC * _NS
_K = 125
_NB = 5
_LA = 2
_RPT = 624
_TAIL = _N - _RPT * _NS
_DDEG = 8

_EPW = _E // _NW
_NCH_D = _EPW // _K
_EPS = _E // _NS
_NCH_E = _EPS // _K

_F32 = jnp.float32
_SC_PARAMS = pltpu.CompilerParams(use_tc_tiling_on_sc=False)


def _mesh():
    return plsc.VectorSubcoreMesh(core_axis_name="c", subcore_axis_name="s")


def _zero_share(acc, zeros, sid):
    rbase = sid * _RPT
    pltpu.sync_copy(zeros.at[pl.ds(rbase, _RPT)], acc.at[pl.ds(rbase, _RPT)])

    @pl.when(sid == _NS - 1)
    def _():
        pltpu.sync_copy(zeros.at[pl.ds(_RPT * _NS, _TAIL)],
                        acc.at[pl.ds(_RPT * _NS, _TAIL)])


def _copy_out(acc, out, cid, sid):
    rbase = sid * _RPT
    pltpu.sync_copy(acc.at[pl.ds(rbase, _RPT)], out.at[cid, pl.ds(rbase, _RPT)])

    @pl.when(sid == _NS - 1)
    def _():
        pltpu.sync_copy(acc.at[pl.ds(_RPT * _NS, _TAIL)],
                        out.at[cid, pl.ds(_RPT * _NS, _TAIL)])


def _degree_body(e0, e1, e2, e3, ones4, zeros, out, acc, idxs,
                 o0, o1, o2, o3, *ssem):
    cid = lax.axis_index("c")
    sid = lax.axis_index("s")
    wid = cid * _NS + sid
    ones = (o0, o1, o2, o3)

    loads = [pltpu.async_copy(e.at[wid], idxs.at[j], ssem[0])
             for j, e in enumerate((e0, e1, e2, e3))]
    loads += [pltpu.async_copy(ones4.at[j], ones[j], ssem[1])
              for j in range(4)]
    _zero_share(acc, zeros, sid)
    for cp in loads:
        cp.wait()
    plsc.subcore_barrier()

    for j in range(4):
        def group(g, carry):
            for b in range(_NB):
                c = g * _NB + b
                if j == 0:
                    @pl.when(c >= _NB)
                    def _():
                        pltpu.make_async_copy(out.at[0, pl.ds(0, _K)],
                                              ones[0], ssem[b]).wait()
                else:
                    pltpu.make_async_copy(out.at[0, pl.ds(0, _K)],
                                          ones[0], ssem[b]).wait()
                pltpu.async_copy(ones[j], acc.at[idxs.at[j, c]], ssem[b],
                                 add=True)
            return carry

        lax.fori_loop(0, _NCH_D // _NB, group, 0)

    for b in range(_NB):
        pltpu.make_async_copy(out.at[0, pl.ds(0, _K)], ones[0], ssem[b]).wait()

    plsc.subcore_barrier()
    _copy_out(acc, out, cid, sid)


def _make_degrees():
    return pl.kernel(
        _degree_body,
        out_type=jax.ShapeDtypeStruct((_NC, _N, _DDEG), _F32),
        mesh=_mesh(),
        compiler_params=_SC_PARAMS,
        scratch_types=(
            [pltpu.VMEM_SHARED((_N, _DDEG), _F32),
             pltpu.VMEM((4, _NCH_D, _K), jnp.int32)]
            + [pltpu.VMEM((_K, _DDEG), _F32) for _ in range(4)]
            + [pltpu.SemaphoreType.DMA for _ in range(_NB)]
        ),
    )


def _make_edge_pass(dh, mode):
    nch = _NCH_E if mode == "split" else _NCH_D

    def body(table, srcx, dstx, zeros, out, acc, sidx, didx, *rest):
        rows = rest[0:_NB]
        gsem = rest[_NB:2 * _NB]
        ssem = rest[2 * _NB:3 * _NB]
        cid = lax.axis_index("c")
        sid = lax.axis_index("s")

        if mode == "split":
            ld_s = pltpu.async_copy(srcx.at[cid, sid], sidx, gsem[1])
            ld_d = pltpu.async_copy(dstx.at[sid], didx, gsem[2])
        else:
            wid = cid * _NS + sid
            ld_s = pltpu.async_copy(srcx.at[wid], sidx, gsem[1])
            ld_d = pltpu.async_copy(dstx.at[wid], didx, gsem[2])
        _zero_share(acc, zeros, sid)
        ld_s.wait()
        ld_d.wait()
        plsc.subcore_barrier()

        for c in range(_LA):
            pltpu.async_copy(table.at[sidx.at[c]], rows[c % _NB],
                             gsem[c % _NB])

        def group(g, carry):
            for b in range(_NB):
                c = g * _NB + b
                bg = (b + _LA) % _NB
                pltpu.make_async_copy(table.at[pl.ds(0, _K)], rows[b],
                                      gsem[b]).wait()
                pltpu.async_copy(rows[b], acc.at[didx.at[c]], ssem[b],
                                 add=True)

                @pl.when(jnp.logical_and(c + _LA < nch, c + _LA >= _NB))
                def _():
                    pltpu.make_async_copy(table.at[pl.ds(0, _K)], rows[bg],
                                          ssem[bg]).wait()

                @pl.when(c + _LA < nch)
                def _():
                    pltpu.async_copy(table.at[sidx.at[c + _LA]], rows[bg],
                                     gsem[bg])
            return carry

        lax.fori_loop(0, nch // _NB, group, 0)

        for b in range(_NB):
            pltpu.make_async_copy(table.at[pl.ds(0, _K)], rows[b],
                                  ssem[b]).wait()

        plsc.subcore_barrier()
        rbase = sid * _RPT
        if mode == "split":
            dsts = (out.at[pl.ds(rbase, _RPT), pl.ds(cid * dh, dh)],
                    out.at[pl.ds(_RPT * _NS, _TAIL), pl.ds(cid * dh, dh)])
        else:
            dsts = (out.at[cid, pl.ds(rbase, _RPT), pl.ds(0, dh)],
                    out.at[cid, pl.ds(_RPT * _NS, _TAIL), pl.ds(0, dh)])
        pltpu.sync_copy(acc.at[pl.ds(rbase, _RPT)], dsts[0])

        @pl.when(sid == _NS - 1)
        def _():
            pltpu.sync_copy(acc.at[pl.ds(_RPT * _NS, _TAIL)], dsts[1])

    out_shape = ((_N, 2 * dh) if mode == "split" else (_NC, _N, 128))
    return pl.kernel(
        body,
        out_type=jax.ShapeDtypeStruct(out_shape, _F32),
        mesh=_mesh(),
        compiler_params=_SC_PARAMS,
        scratch_types=(
            [pltpu.VMEM_SHARED((_N, dh), _F32),
             pltpu.VMEM((nch, _K), jnp.int32),
             pltpu.VMEM((nch, _K), jnp.int32)]
            + [pltpu.VMEM((_K, dh), _F32) for _ in range(_NB)]
            + [pltpu.SemaphoreType.DMA for _ in range(2 * _NB)]
        ),
    )


def _rs(cnt, j):
    return lax.rsqrt(jnp.maximum(cnt[:, j:j + 1], 1.0))


def _cnts(cnt_ref):
    return cnt_ref[0] + cnt_ref[1]


def _tc_z_body(x_ref, w_ref, z_ref):
    z_ref[...] = jnp.dot(x_ref[...], w_ref[...], preferred_element_type=_F32)


def _tc_pre_body(z_ref, cnt_ref, y_ref):
    cnt = _cnts(cnt_ref)
    y_ref[...] = z_ref[...] * _rs(cnt, 0)


def _tc_mid_body(s1_ref, cnt_ref, b1_ref, w2_ref, y2_ref):
    cnt = _cnts(cnt_ref)
    h = jnp.maximum(s1_ref[...] * _rs(cnt, 1) + b1_ref[...], 0.0)
    y2 = jnp.dot(h * _rs(cnt, 2), w2_ref[...], preferred_element_type=_F32)
    y2_ref[:, pl.ds(0, y2.shape[1])] = y2


def _tc_post_body(s2_ref, cnt_ref, b2_ref, out_ref):
    cnt = _cnts(cnt_ref)
    d = out_ref.shape[1]
    s2 = s2_ref[0][:, :d] + s2_ref[1][:, :d]
    out_ref[...] = s2 * _rs(cnt, 3) + b2_ref[...]


def _split_edges(edge_index):
    src = edge_index[0].reshape(_NS, _NCH_E, _K)
    dst = edge_index[1].reshape(_NS, _NCH_E, _K)
    src4 = jnp.stack([2 * src, 2 * src + 1])
    return src4, dst


def _gcn(inputs, edge_index0, edge_index1, W1, b1, W2, b2):
    src0_d = edge_index0[0].reshape(_NW, _NCH_D, _K)
    dst0_d = edge_index0[1].reshape(_NW, _NCH_D, _K)
    src1_d = edge_index1[0].reshape(_NW, _NCH_D, _K)
    dst1_d = edge_index1[1].reshape(_NW, _NCH_D, _K)
    d_hid = W1.shape[1]
    d_out = W2.shape[1]
    src0, dst0 = _split_edges(edge_index0)
    src1f = src1_d * 2
    dst1f = dst1_d

    ones4 = jnp.broadcast_to(
        (jnp.arange(_DDEG)[None, None, :] ==
         jnp.arange(4)[:, None, None]).astype(_F32), (4, _K, _DDEG))
    zeros_deg = jnp.zeros((_N, _DDEG), _F32)
    zeros_edge = jnp.zeros((_N, d_hid // 2), _F32)

    cnt = _make_degrees()(src0_d, dst0_d, src1_d, dst1_d, ones4, zeros_deg)
    cntv = cnt

    z = pl.pallas_call(
        _tc_z_body,
        out_shape=jax.ShapeDtypeStruct((_N, d_hid), _F32),
    )(inputs, W1)

    y1 = pl.pallas_call(
        _tc_pre_body,
        out_shape=jax.ShapeDtypeStruct((_N, d_hid), _F32),
    )(z, cntv)

    s1 = _make_edge_pass(d_hid // 2, "split")(
        y1.reshape(2 * _N, d_hid // 2), src0, dst0, zeros_edge)

    y2 = pl.pallas_call(
        _tc_mid_body,
        out_shape=jax.ShapeDtypeStruct((_N, d_hid), _F32),
    )(s1, cntv, b1.reshape(1, -1), W2)

    s2 = _make_edge_pass(d_out, "full")(
        y2.reshape(2 * _N, d_hid // 2), src1f, dst1f, zeros_edge)

    out = pl.pallas_call(
        _tc_post_body,
        out_shape=jax.ShapeDtypeStruct((_N, d_out), _F32),
    )(s2, cntv, b2.reshape(1, -1))

    return out


def kernel(inputs, edge_index0, edge_index1, W1, b1, W2, b2):
    return _gcn(inputs, edge_index0, edge_index1, W1, b1, W2, b2)
